# baseline (speedup 1.0000x reference)
import functools

import jax
import jax.numpy as jnp
from jax import lax
from jax.experimental import pallas as pl
from jax.experimental.pallas import tpu as pltpu
from jax.experimental.pallas import tpu_sc as plsc

_SEQ = 2048
_NOUT = 1000
_NIN = 1000
_NCORES = 1
_NSUB = 16
_NW = _NCORES * _NSUB
_CHUNK = _SEQ // _NW
_LANES = 16
_BT = 128


def _sc_gather_scan_body(seq_hbm, delta_hbm, g_hbm, tots_hbm,
                         seq_v, delta_v, g_v, pub_v, sem1, sem2):
    wid = lax.axis_index("s") * _NCORES + lax.axis_index("c")
    base = wid * _CHUNK
    h1 = pltpu.async_copy(seq_hbm.at[pl.ds(base, _CHUNK)], seq_v, sem1)
    h2 = pltpu.async_copy(delta_hbm, delta_v, sem2)
    h1.wait()
    h2.wait()

    def gather_scan(j, carry):
        idx = seq_v[pl.ds(j * _LANES, _LANES)]
        vals = plsc.load_gather(delta_v, [idx])
        g_v[pl.ds(j * _LANES, _LANES)] = plsc.cumsum(vals) + carry
        return carry + jnp.sum(vals)

    tot = lax.fori_loop(0, _CHUNK // _LANES, gather_scan, jnp.float32(0.0))

    pub_v[...] = jnp.broadcast_to(tot, (_LANES,))
    pltpu.sync_copy(g_v, g_hbm.at[pl.ds(base, _CHUNK)])
    pltpu.sync_copy(pub_v, tots_hbm.at[wid])


def _sc_gather_scan(input_seq, delta):
    mesh = plsc.VectorSubcoreMesh(
        core_axis_name="c", subcore_axis_name="s", num_cores=_NCORES)
    run = functools.partial(
        pl.kernel,
        mesh=mesh,
        compiler_params=pltpu.CompilerParams(needs_layout_passes=False),
        out_type=[jax.ShapeDtypeStruct((_SEQ,), jnp.float32),
                  jax.ShapeDtypeStruct((_NW, _LANES), jnp.float32)],
        scratch_types=[
            pltpu.VMEM((_CHUNK,), jnp.int32),
            pltpu.VMEM((_NIN,), jnp.float32),
            pltpu.VMEM((_CHUNK,), jnp.float32),
            pltpu.VMEM((_LANES,), jnp.float32),
            pltpu.SemaphoreType.DMA,
            pltpu.SemaphoreType.DMA,
        ],
    )(_sc_gather_scan_body)
    return run(input_seq, delta)


def _softmax_t_body(g_ref, tot_ref, w_ref, b_ref, o_ref):
    i = pl.program_id(0)
    tcol = tot_ref[...][:, 0:1]
    t_glob = lax.broadcasted_iota(jnp.int32, (_NW, _BT), 1) + i * _BT
    k = lax.broadcasted_iota(jnp.int32, (_NW, _BT), 0)
    mask = (t_glob >= (k + 1) * _CHUNK).astype(jnp.float32)
    offs = jnp.sum(tcol * mask, axis=0, keepdims=True)
    c = g_ref[...] + offs
    logits = w_ref[...] * c + b_ref[...]
    m = jnp.max(logits, axis=0, keepdims=True)
    e = jnp.exp(logits - m)
    s = jnp.sum(e, axis=0, keepdims=True)
    o_ref[...] = e / s


def _tc_softmax_t(g_row, tots, w_col, b_col):
    return pl.pallas_call(
        _softmax_t_body,
        grid=(_SEQ // _BT,),
        in_specs=[
            pl.BlockSpec((1, _BT), lambda i: (0, i)),
            pl.BlockSpec((_NW, _LANES), lambda i: (0, 0)),
            pl.BlockSpec((_NOUT, 1), lambda i: (0, 0)),
            pl.BlockSpec((_NOUT, 1), lambda i: (0, 0)),
        ],
        out_specs=pl.BlockSpec((_NOUT, _BT), lambda i: (0, i)),
        out_shape=jax.ShapeDtypeStruct((_NOUT, _SEQ), jnp.float32),
    )(g_row, tots, w_col, b_col)


def kernel(input_seq, delta, W, b):
    g, tots = _sc_gather_scan(input_seq, delta)
    out_t = _tc_softmax_t(g[None, :], tots, W, b[:, None])
    return out_t.T


# speedup vs baseline: 1.2486x; 1.2486x over previous
import functools

import jax
import jax.numpy as jnp
from jax import lax
from jax.experimental import pallas as pl
from jax.experimental.pallas import tpu as pltpu
from jax.experimental.pallas import tpu_sc as plsc

_SEQ = 2048
_NOUT = 1000
_NIN = 1000
_NCORES = 1
_NSUB = 16
_NW = _NCORES * _NSUB
_CHUNK = _SEQ // _NW
_LANES = 16
_BT = 512


def _sc_gather_scan_body(seq_hbm, delta_hbm, g_hbm, tots_hbm,
                         seq_v, delta_v, g_v, pub_v, sem1, sem2):
    wid = lax.axis_index("s") * _NCORES + lax.axis_index("c")
    base = wid * _CHUNK
    h1 = pltpu.async_copy(seq_hbm.at[pl.ds(base, _CHUNK)], seq_v, sem1)
    h2 = pltpu.async_copy(delta_hbm, delta_v, sem2)
    h1.wait()
    h2.wait()

    def gather_scan(j, carry):
        idx = seq_v[pl.ds(j * _LANES, _LANES)]
        vals = plsc.load_gather(delta_v, [idx])
        g_v[pl.ds(j * _LANES, _LANES)] = plsc.cumsum(vals) + carry
        return carry + jnp.sum(vals)

    tot = lax.fori_loop(0, _CHUNK // _LANES, gather_scan, jnp.float32(0.0))

    pub_v[...] = jnp.broadcast_to(tot, (_LANES,))
    pltpu.sync_copy(g_v, g_hbm.at[pl.ds(base, _CHUNK)])
    pltpu.sync_copy(pub_v, tots_hbm.at[wid])


def _sc_gather_scan(input_seq, delta):
    mesh = plsc.VectorSubcoreMesh(
        core_axis_name="c", subcore_axis_name="s", num_cores=_NCORES)
    run = functools.partial(
        pl.kernel,
        mesh=mesh,
        compiler_params=pltpu.CompilerParams(needs_layout_passes=False),
        out_type=[jax.ShapeDtypeStruct((_SEQ,), jnp.float32),
                  jax.ShapeDtypeStruct((_NW, _LANES), jnp.float32)],
        scratch_types=[
            pltpu.VMEM((_CHUNK,), jnp.int32),
            pltpu.VMEM((_NIN,), jnp.float32),
            pltpu.VMEM((_CHUNK,), jnp.float32),
            pltpu.VMEM((_LANES,), jnp.float32),
            pltpu.SemaphoreType.DMA,
            pltpu.SemaphoreType.DMA,
        ],
    )(_sc_gather_scan_body)
    return run(input_seq, delta)


def _softmax_t_body(g_ref, tot_ref, w_ref, b_ref, o_ref):
    i = pl.program_id(0)
    tcol = tot_ref[...][:, 0:1]
    t_glob = lax.broadcasted_iota(jnp.int32, (_NW, _BT), 1) + i * _BT
    k = lax.broadcasted_iota(jnp.int32, (_NW, _BT), 0)
    mask = (t_glob >= (k + 1) * _CHUNK).astype(jnp.float32)
    offs = jnp.sum(tcol * mask, axis=0, keepdims=True)
    c = g_ref[...] + offs
    logits = w_ref[...] * c + b_ref[...]
    m = jnp.max(logits, axis=0, keepdims=True)
    e = jnp.exp(logits - m)
    s = jnp.sum(e, axis=0, keepdims=True)
    o_ref[...] = e / s


def _tc_softmax_t(g_row, tots, w_col, b_col):
    return pl.pallas_call(
        _softmax_t_body,
        grid=(_SEQ // _BT,),
        in_specs=[
            pl.BlockSpec((1, _BT), lambda i: (0, i)),
            pl.BlockSpec((_NW, _LANES), lambda i: (0, 0)),
            pl.BlockSpec((_NOUT, 1), lambda i: (0, 0)),
            pl.BlockSpec((_NOUT, 1), lambda i: (0, 0)),
        ],
        out_specs=pl.BlockSpec((_NOUT, _BT), lambda i: (0, i)),
        out_shape=jax.ShapeDtypeStruct((_NOUT, _SEQ), jnp.float32),
    )(g_row, tots, w_col, b_col)


def kernel(input_seq, delta, W, b):
    g, tots = _sc_gather_scan(input_seq, delta)
    out_t = _tc_softmax_t(g[None, :], tots, W, b[:, None])
    return out_t.T


# speedup vs baseline: 1.2524x; 1.0030x over previous
import functools

import jax
import jax.numpy as jnp
from jax import lax
from jax.experimental import pallas as pl
from jax.experimental.pallas import tpu as pltpu
from jax.experimental.pallas import tpu_sc as plsc

_SEQ = 2048
_NOUT = 1000
_NIN = 1000
_NCORES = 1
_NSUB = 16
_NW = _NCORES * _NSUB
_CHUNK = _SEQ // _NW
_LANES = 16
_BT = 1024


def _sc_gather_scan_body(seq_hbm, delta_hbm, g_hbm, tots_hbm,
                         seq_v, delta_v, g_v, pub_v, sem1, sem2):
    wid = lax.axis_index("s") * _NCORES + lax.axis_index("c")
    base = wid * _CHUNK
    h1 = pltpu.async_copy(seq_hbm.at[pl.ds(base, _CHUNK)], seq_v, sem1)
    h2 = pltpu.async_copy(delta_hbm, delta_v, sem2)
    h1.wait()
    h2.wait()

    def gather_scan(j, carry):
        idx = seq_v[pl.ds(j * _LANES, _LANES)]
        vals = plsc.load_gather(delta_v, [idx])
        g_v[pl.ds(j * _LANES, _LANES)] = plsc.cumsum(vals) + carry
        return carry + jnp.sum(vals)

    tot = lax.fori_loop(0, _CHUNK // _LANES, gather_scan, jnp.float32(0.0))

    pub_v[...] = jnp.broadcast_to(tot, (_LANES,))
    pltpu.sync_copy(g_v, g_hbm.at[pl.ds(base, _CHUNK)])
    pltpu.sync_copy(pub_v, tots_hbm.at[wid])


def _sc_gather_scan(input_seq, delta):
    mesh = plsc.VectorSubcoreMesh(
        core_axis_name="c", subcore_axis_name="s", num_cores=_NCORES)
    run = functools.partial(
        pl.kernel,
        mesh=mesh,
        compiler_params=pltpu.CompilerParams(needs_layout_passes=False),
        out_type=[jax.ShapeDtypeStruct((_SEQ,), jnp.float32),
                  jax.ShapeDtypeStruct((_NW, _LANES), jnp.float32)],
        scratch_types=[
            pltpu.VMEM((_CHUNK,), jnp.int32),
            pltpu.VMEM((_NIN,), jnp.float32),
            pltpu.VMEM((_CHUNK,), jnp.float32),
            pltpu.VMEM((_LANES,), jnp.float32),
            pltpu.SemaphoreType.DMA,
            pltpu.SemaphoreType.DMA,
        ],
    )(_sc_gather_scan_body)
    return run(input_seq, delta)


def _softmax_t_body(g_ref, tot_ref, w_ref, b_ref, o_ref):
    i = pl.program_id(0)
    tcol = tot_ref[...][:, 0:1]
    t_glob = lax.broadcasted_iota(jnp.int32, (_NW, _BT), 1) + i * _BT
    k = lax.broadcasted_iota(jnp.int32, (_NW, _BT), 0)
    mask = (t_glob >= (k + 1) * _CHUNK).astype(jnp.float32)
    offs = jnp.sum(tcol * mask, axis=0, keepdims=True)
    c = g_ref[...] + offs
    logits = w_ref[...] * c + b_ref[...]
    m = jnp.max(logits, axis=0, keepdims=True)
    e = jnp.exp(logits - m)
    s = jnp.sum(e, axis=0, keepdims=True)
    o_ref[...] = e / s


def _tc_softmax_t(g_row, tots, w_col, b_col):
    return pl.pallas_call(
        _softmax_t_body,
        grid=(_SEQ // _BT,),
        in_specs=[
            pl.BlockSpec((1, _BT), lambda i: (0, i)),
            pl.BlockSpec((_NW, _LANES), lambda i: (0, 0)),
            pl.BlockSpec((_NOUT, 1), lambda i: (0, 0)),
            pl.BlockSpec((_NOUT, 1), lambda i: (0, 0)),
        ],
        out_specs=pl.BlockSpec((_NOUT, _BT), lambda i: (0, i)),
        out_shape=jax.ShapeDtypeStruct((_NOUT, _SEQ), jnp.float32),
    )(g_row, tots, w_col, b_col)


def kernel(input_seq, delta, W, b):
    g, tots = _sc_gather_scan(input_seq, delta)
    out_t = _tc_softmax_t(g[None, :], tots, W, b[:, None])
    return out_t.T


# speedup vs baseline: 1.2553x; 1.0024x over previous
import functools

import jax
import jax.numpy as jnp
from jax import lax
from jax.experimental import pallas as pl
from jax.experimental.pallas import tpu as pltpu
from jax.experimental.pallas import tpu_sc as plsc

_SEQ = 2048
_NOUT = 1000
_NIN = 1000
_NCORES = 1
_NSUB = 16
_NW = _NCORES * _NSUB
_CHUNK = _SEQ // _NW
_LANES = 16
_BT = 1024


def _sc_gather_scan_body(seq_hbm, delta_hbm, g_hbm, tots_hbm,
                         seq_v, delta_v, g_v, pub_v, sem1, sem2):
    wid = lax.axis_index("s") * _NCORES + lax.axis_index("c")
    base = wid * _CHUNK
    h1 = pltpu.async_copy(seq_hbm.at[pl.ds(base, _CHUNK)], seq_v, sem1)
    h2 = pltpu.async_copy(delta_hbm, delta_v, sem2)
    h1.wait()
    h2.wait()

    def gather_scan(j, carry):
        idx = seq_v[pl.ds(j * _LANES, _LANES)]
        vals = plsc.load_gather(delta_v, [idx])
        g_v[pl.ds(j * _LANES, _LANES)] = plsc.cumsum(vals) + carry
        return carry + jnp.sum(vals)

    tot = lax.fori_loop(0, _CHUNK // _LANES, gather_scan, jnp.float32(0.0))

    pub_v[...] = jnp.broadcast_to(tot, (_LANES,))
    pltpu.sync_copy(g_v, g_hbm.at[pl.ds(base, _CHUNK)])
    pltpu.sync_copy(pub_v, tots_hbm.at[wid])


def _sc_gather_scan(input_seq, delta):
    mesh = plsc.VectorSubcoreMesh(
        core_axis_name="c", subcore_axis_name="s", num_cores=_NCORES)
    run = functools.partial(
        pl.kernel,
        mesh=mesh,
        compiler_params=pltpu.CompilerParams(needs_layout_passes=False),
        out_type=[jax.ShapeDtypeStruct((_SEQ,), jnp.float32),
                  jax.ShapeDtypeStruct((_NW, _LANES), jnp.float32)],
        scratch_types=[
            pltpu.VMEM((_CHUNK,), jnp.int32),
            pltpu.VMEM((_NIN,), jnp.float32),
            pltpu.VMEM((_CHUNK,), jnp.float32),
            pltpu.VMEM((_LANES,), jnp.float32),
            pltpu.SemaphoreType.DMA,
            pltpu.SemaphoreType.DMA,
        ],
    )(_sc_gather_scan_body)
    return run(input_seq, delta)


def _softmax_t_body(g_ref, tot_ref, w_ref, b_ref, o_ref):
    i = pl.program_id(0)
    tcol = tot_ref[...][:, 0:1]
    t_glob = lax.broadcasted_iota(jnp.int32, (_NW, _BT), 1) + i * _BT
    k = lax.broadcasted_iota(jnp.int32, (_NW, _BT), 0)
    mask = (t_glob >= (k + 1) * _CHUNK).astype(jnp.float32)
    offs = jnp.sum(tcol * mask, axis=0, keepdims=True)
    c = g_ref[...] + offs
    logits = w_ref[...] * c + b_ref[...]
    m = jnp.max(logits, axis=0, keepdims=True)
    e = jnp.exp(logits - m)
    s = jnp.sum(e, axis=0, keepdims=True)
    o_ref[...] = e * (1.0 / s)


def _tc_softmax_t(g_row, tots, w_col, b_col):
    return pl.pallas_call(
        _softmax_t_body,
        grid=(_SEQ // _BT,),
        in_specs=[
            pl.BlockSpec((1, _BT), lambda i: (0, i)),
            pl.BlockSpec((_NW, _LANES), lambda i: (0, 0)),
            pl.BlockSpec((_NOUT, 1), lambda i: (0, 0)),
            pl.BlockSpec((_NOUT, 1), lambda i: (0, 0)),
        ],
        out_specs=pl.BlockSpec((_NOUT, _BT), lambda i: (0, i)),
        out_shape=jax.ShapeDtypeStruct((_NOUT, _SEQ), jnp.float32),
    )(g_row, tots, w_col, b_col)


def kernel(input_seq, delta, W, b):
    g, tots = _sc_gather_scan(input_seq, delta)
    out_t = _tc_softmax_t(g[None, :], tots, W, b[:, None])
    return out_t.T


# speedup vs baseline: 1.2872x; 1.0254x over previous
import functools

import jax
import jax.numpy as jnp
from jax import lax
from jax.experimental import pallas as pl
from jax.experimental.pallas import tpu as pltpu
from jax.experimental.pallas import tpu_sc as plsc

_SEQ = 2048
_NOUT = 1000
_NIN = 1000
_NCORES = 1
_NSUB = 16
_NW = _NCORES * _NSUB
_CHUNK = _SEQ // _NW
_LANES = 16
_BT = 1024


def _sc_gather_scan_body(seq_hbm, delta_hbm, g_hbm, tots_hbm,
                         seq_v, delta_v, g_v, pub_v, sem1, sem2):
    wid = lax.axis_index("s") * _NCORES + lax.axis_index("c")
    base = wid * _CHUNK
    h1 = pltpu.async_copy(seq_hbm.at[pl.ds(base, _CHUNK)], seq_v, sem1)
    h2 = pltpu.async_copy(delta_hbm, delta_v, sem2)
    h1.wait()
    h2.wait()

    def gather_scan(j, carry):
        idx = seq_v[pl.ds(j * _LANES, _LANES)]
        vals = plsc.load_gather(delta_v, [idx])
        g_v[pl.ds(j * _LANES, _LANES)] = plsc.cumsum(vals) + carry
        return carry + jnp.sum(vals)

    tot = lax.fori_loop(0, _CHUNK // _LANES, gather_scan, jnp.float32(0.0))

    pub_v[...] = jnp.broadcast_to(tot, (_LANES,))
    pltpu.sync_copy(g_v, g_hbm.at[pl.ds(base, _CHUNK)])
    pltpu.sync_copy(pub_v, tots_hbm.at[wid])


def _sc_gather_scan(input_seq, delta):
    mesh = plsc.VectorSubcoreMesh(
        core_axis_name="c", subcore_axis_name="s", num_cores=_NCORES)
    run = functools.partial(
        pl.kernel,
        mesh=mesh,
        compiler_params=pltpu.CompilerParams(needs_layout_passes=False),
        out_type=[jax.ShapeDtypeStruct((_SEQ,), jnp.float32),
                  jax.ShapeDtypeStruct((_NW, _LANES), jnp.float32)],
        scratch_types=[
            pltpu.VMEM((_CHUNK,), jnp.int32),
            pltpu.VMEM((_NIN,), jnp.float32),
            pltpu.VMEM((_CHUNK,), jnp.float32),
            pltpu.VMEM((_LANES,), jnp.float32),
            pltpu.SemaphoreType.DMA,
            pltpu.SemaphoreType.DMA,
        ],
    )(_sc_gather_scan_body)
    return run(input_seq, delta)


def _softmax_t_body(g_ref, tot_ref, w_ref, b_ref, o_ref):
    i = pl.program_id(0)
    tcol = tot_ref[...][:, 0:1]
    t_glob = lax.broadcasted_iota(jnp.int32, (_NW, _BT), 1) + i * _BT
    k = lax.broadcasted_iota(jnp.int32, (_NW, _BT), 0)
    mask = (t_glob >= (k + 1) * _CHUNK).astype(jnp.float32)
    offs = jnp.sum(tcol * mask, axis=0, keepdims=True)
    c = g_ref[...] + offs
    w = w_ref[...]
    b_col = b_ref[...]
    wmax = jnp.max(w)
    wmin = jnp.min(w)
    bmax = jnp.max(b_col)
    m = jnp.maximum(c * wmax, c * wmin) + bmax
    e = jnp.exp(w * c + (b_col - m))
    s = jnp.sum(e, axis=0, keepdims=True)
    o_ref[...] = e * (1.0 / s)


def _tc_softmax_t(g_row, tots, w_col, b_col):
    return pl.pallas_call(
        _softmax_t_body,
        grid=(_SEQ // _BT,),
        in_specs=[
            pl.BlockSpec((1, _BT), lambda i: (0, i)),
            pl.BlockSpec((_NW, _LANES), lambda i: (0, 0)),
            pl.BlockSpec((_NOUT, 1), lambda i: (0, 0)),
            pl.BlockSpec((_NOUT, 1), lambda i: (0, 0)),
        ],
        out_specs=pl.BlockSpec((_NOUT, _BT), lambda i: (0, i)),
        out_shape=jax.ShapeDtypeStruct((_NOUT, _SEQ), jnp.float32),
    )(g_row, tots, w_col, b_col)


def kernel(input_seq, delta, W, b):
    g, tots = _sc_gather_scan(input_seq, delta)
    out_t = _tc_softmax_t(g[None, :], tots, W, b[:, None])
    return out_t.T
